# Initial kernel scaffold; baseline (speedup 1.0000x reference)
import jax
import jax.numpy as jnp
from jax.experimental import pallas as pl


def kernel(x, edge_index, edge_attr, w1, b1, w2, b2, w3, b3, w4, b4):
    raise NotImplementedError("write your pallas kernel here")


# speedup vs baseline: 16.5403x; 16.5403x over previous
import functools

import jax
import jax.numpy as jnp
from jax import lax
from jax.experimental import pallas as pl
from jax.experimental.pallas import tpu as pltpu
from jax.experimental.pallas import tpu_sc as plsc

NC = 2
NS = 16
CH = 100
CHP = 104


def _edge_mlp_body(ea_ref, w1k_ref, b1k_ref, w2k_ref, b2k_ref, out_ref):
    h = jnp.dot(ea_ref[...], w1k_ref[...], preferred_element_type=jnp.float32)
    h = jnp.maximum(h + b1k_ref[...], 0.0)
    o = jnp.dot(h, w2k_ref[...], preferred_element_type=jnp.float32)
    out_ref[...] = o + b2k_ref[...]


def _edge_mlp(ea4, w1k, b1k, w2k, b2k):
    rows = ea4.shape[0]
    grid = 25
    r = rows // grid
    return pl.pallas_call(
        _edge_mlp_body,
        grid=(grid,),
        in_specs=[
            pl.BlockSpec((r, 16), lambda i: (i, 0)),
            pl.BlockSpec((16, 64), lambda i: (0, 0)),
            pl.BlockSpec((1, 64), lambda i: (0, 0)),
            pl.BlockSpec((64, 64), lambda i: (0, 0)),
            pl.BlockSpec((1, 64), lambda i: (0, 0)),
        ],
        out_specs=pl.BlockSpec((r, 64), lambda i: (i, 0)),
        out_shape=jax.ShapeDtypeStruct((rows, 64), jnp.float32),
    )(ea4, w1k, b1k, w2k, b2k)


def _make_scatter(n_edges, n_pad):
    n_chunks = n_edges // CH
    cpw = n_chunks // (NC * NS)
    rpt = n_pad // NS
    mesh = plsc.VectorSubcoreMesh(core_axis_name="c", subcore_axis_name="s")

    @functools.partial(
        pl.kernel,
        compiler_params=pltpu.CompilerParams(use_tc_tiling_on_sc=False),
        out_type=jax.ShapeDtypeStruct((NC, 3, n_pad, 16), jnp.float32),
        mesh=mesh,
        scratch_types=[
            pltpu.VMEM((CHP,), jnp.int32),
            pltpu.VMEM((CHP, 16), jnp.float32),
            pltpu.VMEM((CHP, 16), jnp.float32),
            pltpu.VMEM((CHP, 16), jnp.float32),
            pltpu.VMEM_SHARED((n_pad, 16), jnp.float32),
            pltpu.VMEM_SHARED((n_pad, 16), jnp.float32),
            pltpu.VMEM_SHARED((n_pad, 16), jnp.float32),
        ],
    )
    def scatter(seg1, ef3, x0, x1, zrows, out, idx_v, vef, vx0, vx1,
                aef, ax0, ax1):
        c = lax.axis_index("c")
        s = lax.axis_index("s")
        w = c * NS + s
        r0 = s * rpt
        pltpu.sync_copy(zrows.at[pl.ds(0, 8)], vef.at[pl.ds(CH - 4, 8)])
        pltpu.sync_copy(zrows.at[pl.ds(0, 8)], vx0.at[pl.ds(CH - 4, 8)])
        pltpu.sync_copy(zrows.at[pl.ds(0, 8)], vx1.at[pl.ds(CH - 4, 8)])
        pltpu.sync_copy(zrows, aef.at[pl.ds(r0, rpt)])
        pltpu.sync_copy(zrows, ax0.at[pl.ds(r0, rpt)])
        pltpu.sync_copy(zrows, ax1.at[pl.ds(r0, rpt)])
        plsc.subcore_barrier()

        def body(j, carry):
            g = w * cpw + j
            pltpu.sync_copy(seg1.at[pl.ds(g * CHP, CHP)], idx_v)
            pltpu.sync_copy(ef3.at[g], vef.at[pl.ds(0, CH)])
            pltpu.sync_copy(x0.at[g], vx0.at[pl.ds(0, CH)])
            pltpu.sync_copy(x1.at[g], vx1.at[pl.ds(0, CH)])
            pltpu.sync_copy(vef, aef.at[idx_v], add=True)
            pltpu.sync_copy(vx0, ax0.at[idx_v], add=True)
            pltpu.sync_copy(vx1, ax1.at[idx_v], add=True)
            return carry

        lax.fori_loop(0, cpw, body, 0)
        plsc.subcore_barrier()
        pltpu.sync_copy(aef.at[pl.ds(r0, rpt)], out.at[c, 0, pl.ds(r0, rpt)])
        pltpu.sync_copy(ax0.at[pl.ds(r0, rpt)], out.at[c, 1, pl.ds(r0, rpt)])
        pltpu.sync_copy(ax1.at[pl.ds(r0, rpt)], out.at[c, 2, pl.ds(r0, rpt)])

    return scatter


def _ff_body(acc_ref, w3a_ref, w3b_ref, b3_ref, w4_ref, b4_ref, out_ref):
    ef = acc_ref[0, 0] + acc_ref[1, 0]
    x0 = acc_ref[0, 1] + acc_ref[1, 1]
    x1 = acc_ref[0, 2] + acc_ref[1, 2]
    w3a = w3a_ref[...]
    w3b = w3b_ref[...]
    efw = jnp.dot(ef, w3a, preferred_element_type=jnp.float32) + b3_ref[...]
    for b, xb in ((0, x0), (1, x1)):
        h2 = jnp.maximum(
            efw + jnp.dot(xb, w3b, preferred_element_type=jnp.float32), 0.0)
        out_ref[b] = (jnp.dot(h2, w4_ref[...], preferred_element_type=jnp.float32)
                      + b4_ref[...])


def _final_ff(acc, w3a, w3b, b3, w4, b4, n_pad):
    return pl.pallas_call(
        _ff_body,
        in_specs=[
            pl.BlockSpec(memory_space=pltpu.VMEM),
            pl.BlockSpec(memory_space=pltpu.VMEM),
            pl.BlockSpec(memory_space=pltpu.VMEM),
            pl.BlockSpec(memory_space=pltpu.VMEM),
            pl.BlockSpec(memory_space=pltpu.VMEM),
            pl.BlockSpec(memory_space=pltpu.VMEM),
        ],
        out_specs=pl.BlockSpec(memory_space=pltpu.VMEM),
        out_shape=jax.ShapeDtypeStruct((2, n_pad, 32), jnp.float32),
    )(acc, w3a, w3b, b3, w4, b4)


def kernel(x, edge_index, edge_attr, w1, b1, w2, b2, w3, b3, w4, b4):
    B, E, DX = x.shape
    n_rec = 10000
    n_pad = 10240
    eye4 = jnp.eye(4, dtype=jnp.float32)
    w1k = jnp.kron(eye4, w1)
    w2k = jnp.kron(eye4, w2)
    b1k = jnp.tile(b1, 4)[None, :]
    b2k = jnp.tile(b2, 4)[None, :]

    ea4 = edge_attr.reshape(E // 4, 16)
    ef4 = _edge_mlp(ea4, w1k, b1k, w2k, b2k)
    ef3 = ef4.reshape(E // CH, CH, 16)

    seg2d = edge_index[1].reshape(E // CH, CH)
    seg1 = jnp.pad(seg2d, ((0, 0), (0, CHP - CH)),
                   constant_values=n_rec).reshape(-1)
    x3 = x.reshape(B, E // CH, CH, DX)
    zrows = jnp.zeros((n_pad // NS, 16), jnp.float32)
    scatter = _make_scatter(E, n_pad)
    acc = scatter(seg1, ef3, x3[0], x3[1], zrows)

    w3a = w3[:16]
    w3b = w3[16:]
    out = _final_ff(acc, w3a, w3b, b3[None, :], w4, b4[None, :], n_pad)
    return out[:, :n_rec, :]


# speedup vs baseline: 61.6158x; 3.7252x over previous
import functools

import jax
import jax.numpy as jnp
from jax import lax
from jax.experimental import pallas as pl
from jax.experimental.pallas import tpu as pltpu
from jax.experimental.pallas import tpu_sc as plsc

NC = 2
NS = 16
CH = 100
G = 5
EB = 12800


def _fmt_mlp_body(eat_ref, xt_ref, w1t_ref, b1c_ref, w2t_ref, b2c_ref,
                  ef_ref, x0_ref, x1_ref):
    ht = jnp.dot(w1t_ref[...], eat_ref[...], preferred_element_type=jnp.float32)
    ht = jnp.maximum(ht + b1c_ref[...], 0.0)
    eft = jnp.dot(w2t_ref[...], ht, preferred_element_type=jnp.float32)
    eft = eft + b2c_ref[...]
    ef_ref[...] = eft.T
    x0_ref[...] = xt_ref[0].T
    x1_ref[...] = xt_ref[1].T


def _fmt_mlp(eat, xt, w1t, b1c, w2t, b2c, n_edges):
    grid = n_edges // EB
    shp = jax.ShapeDtypeStruct((n_edges, 16), jnp.float32)
    return pl.pallas_call(
        _fmt_mlp_body,
        grid=(grid,),
        in_specs=[
            pl.BlockSpec((4, EB), lambda i: (0, i)),
            pl.BlockSpec((2, 16, EB), lambda i: (0, 0, i)),
            pl.BlockSpec((16, 4), lambda i: (0, 0)),
            pl.BlockSpec((16, 1), lambda i: (0, 0)),
            pl.BlockSpec((16, 16), lambda i: (0, 0)),
            pl.BlockSpec((16, 1), lambda i: (0, 0)),
        ],
        out_specs=[
            pl.BlockSpec((EB, 16), lambda i: (i, 0)),
            pl.BlockSpec((EB, 16), lambda i: (i, 0)),
            pl.BlockSpec((EB, 16), lambda i: (i, 0)),
        ],
        out_shape=[shp, shp, shp],
    )(eat, xt, w1t, b1c, w2t, b2c)


def _make_scatter(n_edges, n_pad):
    n_chunks = n_edges // CH
    ngrp = n_chunks // G // (NC * NS)
    pairs = ngrp // 2
    rpt = n_pad // NS
    mesh = plsc.VectorSubcoreMesh(core_axis_name="c", subcore_axis_name="s")

    @functools.partial(
        pl.kernel,
        compiler_params=pltpu.CompilerParams(use_tc_tiling_on_sc=False),
        out_type=jax.ShapeDtypeStruct((NC, 3, n_pad, 16), jnp.float32),
        mesh=mesh,
        scratch_types=[
            pltpu.VMEM((G, CH), jnp.int32),
            pltpu.VMEM((G, CH), jnp.int32),
            pltpu.VMEM((G, CH, 16), jnp.float32),
            pltpu.VMEM((G, CH, 16), jnp.float32),
            pltpu.VMEM((G, CH, 16), jnp.float32),
            pltpu.VMEM((G, CH, 16), jnp.float32),
            pltpu.VMEM((G, CH, 16), jnp.float32),
            pltpu.VMEM((G, CH, 16), jnp.float32),
            pltpu.VMEM_SHARED((n_pad, 16), jnp.float32),
            pltpu.VMEM_SHARED((n_pad, 16), jnp.float32),
            pltpu.VMEM_SHARED((n_pad, 16), jnp.float32),
            pltpu.SemaphoreType.DMA,
            pltpu.SemaphoreType.DMA,
            pltpu.SemaphoreType.DMA,
            pltpu.SemaphoreType.DMA,
        ],
    )
    def scatter(seg5, ef5, x05, x15, zrows, out,
                ixA, ixB, vefA, vx0A, vx1A, vefB, vx0B, vx1B,
                aef, ax0, ax1, rA, rB, sA, sB):
        c = lax.axis_index("c")
        s = lax.axis_index("s")
        w = c * NS + s
        r0 = s * rpt
        pltpu.sync_copy(zrows, aef.at[pl.ds(r0, rpt)])
        pltpu.sync_copy(zrows, ax0.at[pl.ds(r0, rpt)])
        pltpu.sync_copy(zrows, ax1.at[pl.ds(r0, rpt)])
        plsc.subcore_barrier()

        bufsA = ((vefA, ef5, aef), (vx0A, x05, ax0), (vx1A, x15, ax1))
        bufsB = ((vefB, ef5, aef), (vx0B, x05, ax0), (vx1B, x15, ax1))

        def body(i, carry):
            gA = w * ngrp + 2 * i
            gB = gA + 1
            dA = [pltpu.async_copy(seg5.at[gA], ixA, rA)]
            dA += [pltpu.async_copy(src.at[gA], buf, rA)
                   for buf, src, _ in bufsA]
            dB = [pltpu.async_copy(seg5.at[gB], ixB, rB)]
            dB += [pltpu.async_copy(src.at[gB], buf, rB)
                   for buf, src, _ in bufsB]
            for d in dA:
                d.wait()
            sAd = [pltpu.async_copy(buf.at[b], acc.at[ixA.at[b]], sA, add=True)
                   for b in range(G) for buf, _, acc in bufsA]
            for d in dB:
                d.wait()
            sBd = [pltpu.async_copy(buf.at[b], acc.at[ixB.at[b]], sB, add=True)
                   for b in range(G) for buf, _, acc in bufsB]
            for d in sAd:
                d.wait()
            for d in sBd:
                d.wait()
            return carry

        lax.fori_loop(0, pairs, body, 0)
        plsc.subcore_barrier()
        pltpu.sync_copy(aef.at[pl.ds(r0, rpt)], out.at[c, 0, pl.ds(r0, rpt)])
        pltpu.sync_copy(ax0.at[pl.ds(r0, rpt)], out.at[c, 1, pl.ds(r0, rpt)])
        pltpu.sync_copy(ax1.at[pl.ds(r0, rpt)], out.at[c, 2, pl.ds(r0, rpt)])

    return scatter


def _ff_body(acc_ref, w3a_ref, w3b_ref, b3_ref, w4_ref, b4_ref, out_ref):
    ef = acc_ref[0, 0] + acc_ref[1, 0]
    x0 = acc_ref[0, 1] + acc_ref[1, 1]
    x1 = acc_ref[0, 2] + acc_ref[1, 2]
    w3a = w3a_ref[...]
    w3b = w3b_ref[...]
    efw = jnp.dot(ef, w3a, preferred_element_type=jnp.float32) + b3_ref[...]
    for b, xb in ((0, x0), (1, x1)):
        h2 = jnp.maximum(
            efw + jnp.dot(xb, w3b, preferred_element_type=jnp.float32), 0.0)
        out_ref[b] = (jnp.dot(h2, w4_ref[...], preferred_element_type=jnp.float32)
                      + b4_ref[...])


def _final_ff(acc, w3a, w3b, b3, w4, b4, n_pad):
    return pl.pallas_call(
        _ff_body,
        in_specs=[pl.BlockSpec(memory_space=pltpu.VMEM)] * 6,
        out_specs=pl.BlockSpec(memory_space=pltpu.VMEM),
        out_shape=jax.ShapeDtypeStruct((2, n_pad, 32), jnp.float32),
    )(acc, w3a, w3b, b3, w4, b4)


def kernel(x, edge_index, edge_attr, w1, b1, w2, b2, w3, b3, w4, b4):
    B, E, DX = x.shape
    n_rec = 10000
    n_pad = 10240

    eat = edge_attr.T
    xt = jnp.transpose(x, (0, 2, 1))
    ef, x0r, x1r = _fmt_mlp(eat, xt, w1.T, b1[:, None], w2.T, b2[:, None], E)

    nq = E // CH // G
    seg5 = edge_index[1].reshape(nq, G, CH)
    scatter = _make_scatter(E, n_pad)
    zrows = jnp.zeros((n_pad // NS, 16), jnp.float32)
    acc = scatter(seg5,
                  ef.reshape(nq, G, CH, 16),
                  x0r.reshape(nq, G, CH, 16),
                  x1r.reshape(nq, G, CH, 16),
                  zrows)

    w3a = w3[:16]
    w3b = w3[16:]
    out = _final_ff(acc, w3a, w3b, b3[None, :], w4, b4[None, :], n_pad)
    return out[:, :n_rec, :]


# speedup vs baseline: 89.1926x; 1.4476x over previous
import functools

import jax
import jax.numpy as jnp
from jax import lax
from jax.experimental import pallas as pl
from jax.experimental.pallas import tpu as pltpu
from jax.experimental.pallas import tpu_sc as plsc

NC = 2
NS = 16
CH = 100
G = 5
EB = 12800


def _fmt_mlp_body(eat_ref, xt_ref, w1t_ref, b1c_ref, w2t_ref, b2c_ref,
                  ef_ref, x0_ref, x1_ref):
    ht = jnp.dot(w1t_ref[...], eat_ref[...], preferred_element_type=jnp.float32)
    ht = jnp.maximum(ht + b1c_ref[...], 0.0)
    eft = jnp.dot(w2t_ref[...], ht, preferred_element_type=jnp.float32)
    eft = eft + b2c_ref[...]
    def pack(tv):
        t3 = tv.reshape(EB // 8, 8, 16)
        return jnp.concatenate([t3[:, j, :] for j in range(8)], axis=1)

    ef_ref[...] = pack(eft.T)
    x0_ref[...] = pack(xt_ref[0].T)
    x1_ref[...] = pack(xt_ref[1].T)


def _fmt_mlp(eat, xt, w1t, b1c, w2t, b2c, n_edges):
    grid = n_edges // EB
    shp = jax.ShapeDtypeStruct((n_edges // 8, 128), jnp.float32)
    return pl.pallas_call(
        _fmt_mlp_body,
        grid=(grid,),
        in_specs=[
            pl.BlockSpec((4, EB), lambda i: (0, i)),
            pl.BlockSpec((2, 16, EB), lambda i: (0, 0, i)),
            pl.BlockSpec((16, 4), lambda i: (0, 0)),
            pl.BlockSpec((16, 1), lambda i: (0, 0)),
            pl.BlockSpec((16, 16), lambda i: (0, 0)),
            pl.BlockSpec((16, 1), lambda i: (0, 0)),
        ],
        out_specs=[
            pl.BlockSpec((EB // 8, 128), lambda i: (i, 0)),
            pl.BlockSpec((EB // 8, 128), lambda i: (i, 0)),
            pl.BlockSpec((EB // 8, 128), lambda i: (i, 0)),
        ],
        out_shape=[shp, shp, shp],
    )(eat, xt, w1t, b1c, w2t, b2c)


def _make_scatter(n_edges, n_pad):
    n_chunks = n_edges // CH
    ngrp = n_chunks // G // (NC * NS)
    pairs = ngrp // 2
    rpt = n_pad // NS
    mesh = plsc.VectorSubcoreMesh(core_axis_name="c", subcore_axis_name="s")

    @functools.partial(
        pl.kernel,
        compiler_params=pltpu.CompilerParams(use_tc_tiling_on_sc=False),
        out_type=jax.ShapeDtypeStruct((NC, 3, n_pad, 16), jnp.float32),
        mesh=mesh,
        scratch_types=[
            pltpu.VMEM((G, CH), jnp.int32),
            pltpu.VMEM((G, CH), jnp.int32),
            pltpu.VMEM((G, CH, 16), jnp.float32),
            pltpu.VMEM((G, CH, 16), jnp.float32),
            pltpu.VMEM((G, CH, 16), jnp.float32),
            pltpu.VMEM((G, CH, 16), jnp.float32),
            pltpu.VMEM((G, CH, 16), jnp.float32),
            pltpu.VMEM((G, CH, 16), jnp.float32),
            pltpu.VMEM_SHARED((n_pad, 16), jnp.float32),
            pltpu.VMEM_SHARED((n_pad, 16), jnp.float32),
            pltpu.VMEM_SHARED((n_pad, 16), jnp.float32),
            pltpu.SemaphoreType.DMA,
            pltpu.SemaphoreType.DMA,
            pltpu.SemaphoreType.DMA,
            pltpu.SemaphoreType.DMA,
        ],
    )
    def scatter(seg5, ef5, x05, x15, zrows, out,
                ixA, ixB, vefA, vx0A, vx1A, vefB, vx0B, vx1B,
                aef, ax0, ax1, rA, rB, sA, sB):
        c = lax.axis_index("c")
        s = lax.axis_index("s")
        w = c * NS + s
        r0 = s * rpt
        pltpu.sync_copy(zrows, aef.at[pl.ds(r0, rpt)])
        pltpu.sync_copy(zrows, ax0.at[pl.ds(r0, rpt)])
        pltpu.sync_copy(zrows, ax1.at[pl.ds(r0, rpt)])
        plsc.subcore_barrier()

        bufsA = ((vefA, ef5, aef), (vx0A, x05, ax0), (vx1A, x15, ax1))
        bufsB = ((vefB, ef5, aef), (vx0B, x05, ax0), (vx1B, x15, ax1))

        def body(i, carry):
            gA = w * ngrp + 2 * i
            gB = gA + 1
            dA = [pltpu.async_copy(seg5.at[gA], ixA, rA)]
            dA += [pltpu.async_copy(src.at[gA], buf, rA)
                   for buf, src, _ in bufsA]
            dB = [pltpu.async_copy(seg5.at[gB], ixB, rB)]
            dB += [pltpu.async_copy(src.at[gB], buf, rB)
                   for buf, src, _ in bufsB]
            for d in dA:
                d.wait()
            sAd = [pltpu.async_copy(buf.at[b], acc.at[ixA.at[b]], sA, add=True)
                   for b in range(G) for buf, _, acc in bufsA]
            for d in dB:
                d.wait()
            sBd = [pltpu.async_copy(buf.at[b], acc.at[ixB.at[b]], sB, add=True)
                   for b in range(G) for buf, _, acc in bufsB]
            for d in sAd:
                d.wait()
            for d in sBd:
                d.wait()
            return carry

        lax.fori_loop(0, pairs, body, 0)
        plsc.subcore_barrier()
        pltpu.sync_copy(aef.at[pl.ds(r0, rpt)], out.at[c, 0, pl.ds(r0, rpt)])
        pltpu.sync_copy(ax0.at[pl.ds(r0, rpt)], out.at[c, 1, pl.ds(r0, rpt)])
        pltpu.sync_copy(ax1.at[pl.ds(r0, rpt)], out.at[c, 2, pl.ds(r0, rpt)])

    return scatter


def _ff_body(acc_ref, w3a_ref, w3b_ref, b3_ref, w4_ref, b4_ref, out_ref):
    ef = acc_ref[0, 0] + acc_ref[1, 0]
    x0 = acc_ref[0, 1] + acc_ref[1, 1]
    x1 = acc_ref[0, 2] + acc_ref[1, 2]
    w3a = w3a_ref[...]
    w3b = w3b_ref[...]
    efw = jnp.dot(ef, w3a, preferred_element_type=jnp.float32) + b3_ref[...]
    for b, xb in ((0, x0), (1, x1)):
        h2 = jnp.maximum(
            efw + jnp.dot(xb, w3b, preferred_element_type=jnp.float32), 0.0)
        out_ref[b] = (jnp.dot(h2, w4_ref[...], preferred_element_type=jnp.float32)
                      + b4_ref[...])


def _final_ff(acc, w3a, w3b, b3, w4, b4, n_pad):
    return pl.pallas_call(
        _ff_body,
        in_specs=[pl.BlockSpec(memory_space=pltpu.VMEM)] * 6,
        out_specs=pl.BlockSpec(memory_space=pltpu.VMEM),
        out_shape=jax.ShapeDtypeStruct((2, n_pad, 32), jnp.float32),
    )(acc, w3a, w3b, b3, w4, b4)


def kernel(x, edge_index, edge_attr, w1, b1, w2, b2, w3, b3, w4, b4):
    B, E, DX = x.shape
    n_rec = 10000
    n_pad = 10240

    eat = edge_attr.T
    xt = jnp.transpose(x, (0, 2, 1))
    ef, x0r, x1r = _fmt_mlp(eat, xt, w1.T, b1[:, None], w2.T, b2[:, None], E)

    nq = E // CH // G
    seg5 = edge_index[1].reshape(nq, G, CH)
    scatter = _make_scatter(E, n_pad)
    zrows = jnp.zeros((n_pad // NS, 16), jnp.float32)
    acc = scatter(seg5,
                  ef.reshape(nq, G, CH, 16),
                  x0r.reshape(nq, G, CH, 16),
                  x1r.reshape(nq, G, CH, 16),
                  zrows)

    w3a = w3[:16]
    w3b = w3[16:]
    out = _final_ff(acc, w3a, w3b, b3[None, :], w4, b4[None, :], n_pad)
    return out[:, :n_rec, :]


# speedup vs baseline: 133.5581x; 1.4974x over previous
import functools

import jax
import jax.numpy as jnp
from jax import lax
from jax.experimental import pallas as pl
from jax.experimental.pallas import tpu as pltpu
from jax.experimental.pallas import tpu_sc as plsc

NC = 2
NS = 16
CH = 100
G = 5
EB = 12800


def _fmt_mlp_body(eat_ref, xt_ref, w1t_ref, b1c_ref, w2t_ref, b2c_ref,
                  eye_ref, ef_ref, x0_ref, x1_ref):
    ht = jnp.dot(w1t_ref[...], eat_ref[...], preferred_element_type=jnp.float32)
    ht = jnp.maximum(ht + b1c_ref[...], 0.0)
    eft = jnp.dot(w2t_ref[...], ht, preferred_element_type=jnp.float32)
    eft = eft + b2c_ref[...]
    eye = eye_ref[...]
    br = EB // 8

    def pack_store(tv16, out_ref):
        acc = None
        for j in range(8):
            p = jax.lax.dot_general(
                tv16[:, j * br:(j + 1) * br], eye[16 * j:16 * (j + 1), :],
                (((0,), (0,)), ((), ())), preferred_element_type=jnp.float32)
            acc = p if acc is None else acc + p
        out_ref[...] = acc

    pack_store(eft, ef_ref)
    pack_store(xt_ref[0], x0_ref)
    pack_store(xt_ref[1], x1_ref)


def _fmt_mlp(eat, xt, w1t, b1c, w2t, b2c, n_edges):
    grid = n_edges // EB
    shp = jax.ShapeDtypeStruct((n_edges // 8, 128), jnp.float32)
    return pl.pallas_call(
        _fmt_mlp_body,
        grid=(grid,),
        in_specs=[
            pl.BlockSpec((4, EB), lambda i: (0, i)),
            pl.BlockSpec((2, 16, EB), lambda i: (0, 0, i)),
            pl.BlockSpec((16, 4), lambda i: (0, 0)),
            pl.BlockSpec((16, 1), lambda i: (0, 0)),
            pl.BlockSpec((16, 16), lambda i: (0, 0)),
            pl.BlockSpec((16, 1), lambda i: (0, 0)),
            pl.BlockSpec((128, 128), lambda i: (0, 0)),
        ],
        out_specs=[
            pl.BlockSpec((EB // 8, 128), lambda i: (i, 0)),
            pl.BlockSpec((EB // 8, 128), lambda i: (i, 0)),
            pl.BlockSpec((EB // 8, 128), lambda i: (i, 0)),
        ],
        out_shape=[shp, shp, shp],
    )(eat, xt, w1t, b1c, w2t, b2c, jnp.eye(128, dtype=jnp.float32))


def _make_scatter(n_edges, n_pad):
    n_chunks = n_edges // CH
    ngrp = n_chunks // G // (NC * NS)
    pairs = ngrp // 2
    rpt = n_pad // NS
    mesh = plsc.VectorSubcoreMesh(core_axis_name="c", subcore_axis_name="s")

    @functools.partial(
        pl.kernel,
        compiler_params=pltpu.CompilerParams(use_tc_tiling_on_sc=False),
        out_type=jax.ShapeDtypeStruct((NC, 3, n_pad, 16), jnp.float32),
        mesh=mesh,
        scratch_types=[
            pltpu.VMEM((G, CH), jnp.int32),
            pltpu.VMEM((G, CH), jnp.int32),
            pltpu.VMEM((G, CH, 16), jnp.float32),
            pltpu.VMEM((G, CH, 16), jnp.float32),
            pltpu.VMEM((G, CH, 16), jnp.float32),
            pltpu.VMEM((G, CH, 16), jnp.float32),
            pltpu.VMEM((G, CH, 16), jnp.float32),
            pltpu.VMEM((G, CH, 16), jnp.float32),
            pltpu.VMEM_SHARED((n_pad, 16), jnp.float32),
            pltpu.VMEM_SHARED((n_pad, 16), jnp.float32),
            pltpu.VMEM_SHARED((n_pad, 16), jnp.float32),
            pltpu.SemaphoreType.DMA,
            pltpu.SemaphoreType.DMA,
            pltpu.SemaphoreType.DMA,
            pltpu.SemaphoreType.DMA,
        ],
    )
    def scatter(seg5, ef5, x05, x15, zrows, out,
                ixA, ixB, vefA, vx0A, vx1A, vefB, vx0B, vx1B,
                aef, ax0, ax1, rA, rB, sA, sB):
        c = lax.axis_index("c")
        s = lax.axis_index("s")
        w = c * NS + s
        r0 = s * rpt
        pltpu.sync_copy(zrows, aef.at[pl.ds(r0, rpt)])
        pltpu.sync_copy(zrows, ax0.at[pl.ds(r0, rpt)])
        pltpu.sync_copy(zrows, ax1.at[pl.ds(r0, rpt)])
        plsc.subcore_barrier()

        bufsA = ((vefA, ef5, aef), (vx0A, x05, ax0), (vx1A, x15, ax1))
        bufsB = ((vefB, ef5, aef), (vx0B, x05, ax0), (vx1B, x15, ax1))

        def body(i, carry):
            gA = w * ngrp + 2 * i
            gB = gA + 1
            dA = [pltpu.async_copy(seg5.at[gA], ixA, rA)]
            dA += [pltpu.async_copy(src.at[gA], buf, rA)
                   for buf, src, _ in bufsA]
            dB = [pltpu.async_copy(seg5.at[gB], ixB, rB)]
            dB += [pltpu.async_copy(src.at[gB], buf, rB)
                   for buf, src, _ in bufsB]
            for d in dA:
                d.wait()
            sAd = [pltpu.async_copy(buf.at[b], acc.at[ixA.at[b]], sA, add=True)
                   for b in range(G) for buf, _, acc in bufsA]
            for d in dB:
                d.wait()
            sBd = [pltpu.async_copy(buf.at[b], acc.at[ixB.at[b]], sB, add=True)
                   for b in range(G) for buf, _, acc in bufsB]
            for d in sAd:
                d.wait()
            for d in sBd:
                d.wait()
            return carry

        lax.fori_loop(0, pairs, body, 0)
        plsc.subcore_barrier()
        pltpu.sync_copy(aef.at[pl.ds(r0, rpt)], out.at[c, 0, pl.ds(r0, rpt)])
        pltpu.sync_copy(ax0.at[pl.ds(r0, rpt)], out.at[c, 1, pl.ds(r0, rpt)])
        pltpu.sync_copy(ax1.at[pl.ds(r0, rpt)], out.at[c, 2, pl.ds(r0, rpt)])

    return scatter


def _ff_body(acc_ref, w3a_ref, w3b_ref, b3_ref, w4_ref, b4_ref, out_ref):
    ef = acc_ref[0, 0] + acc_ref[1, 0]
    x0 = acc_ref[0, 1] + acc_ref[1, 1]
    x1 = acc_ref[0, 2] + acc_ref[1, 2]
    w3a = w3a_ref[...]
    w3b = w3b_ref[...]
    efw = jnp.dot(ef, w3a, preferred_element_type=jnp.float32) + b3_ref[...]
    for b, xb in ((0, x0), (1, x1)):
        h2 = jnp.maximum(
            efw + jnp.dot(xb, w3b, preferred_element_type=jnp.float32), 0.0)
        out_ref[b] = (jnp.dot(h2, w4_ref[...], preferred_element_type=jnp.float32)
                      + b4_ref[...])


def _final_ff(acc, w3a, w3b, b3, w4, b4, n_pad):
    return pl.pallas_call(
        _ff_body,
        in_specs=[pl.BlockSpec(memory_space=pltpu.VMEM)] * 6,
        out_specs=pl.BlockSpec(memory_space=pltpu.VMEM),
        out_shape=jax.ShapeDtypeStruct((2, n_pad, 32), jnp.float32),
    )(acc, w3a, w3b, b3, w4, b4)


def kernel(x, edge_index, edge_attr, w1, b1, w2, b2, w3, b3, w4, b4):
    B, E, DX = x.shape
    n_rec = 10000
    n_pad = 10240

    eat = edge_attr.T
    xt = jnp.transpose(x, (0, 2, 1))
    ef, x0r, x1r = _fmt_mlp(eat, xt, w1.T, b1[:, None], w2.T, b2[:, None], E)

    nq = E // CH // G
    seg_pk = edge_index[1].reshape(E // EB, 8, EB // 8)
    seg_pk = seg_pk.transpose(0, 2, 1).reshape(-1)
    seg5 = seg_pk.reshape(nq, G, CH)
    scatter = _make_scatter(E, n_pad)
    zrows = jnp.zeros((n_pad // NS, 16), jnp.float32)
    acc = scatter(seg5,
                  ef.reshape(nq, G, CH, 16),
                  x0r.reshape(nq, G, CH, 16),
                  x1r.reshape(nq, G, CH, 16),
                  zrows)

    w3a = w3[:16]
    w3b = w3[16:]
    out = _final_ff(acc, w3a, w3b, b3[None, :], w4, b4[None, :], n_pad)
    return out[:, :n_rec, :]


# speedup vs baseline: 136.0236x; 1.0185x over previous
import functools

import jax
import jax.numpy as jnp
from jax import lax
from jax.experimental import pallas as pl
from jax.experimental.pallas import tpu as pltpu
from jax.experimental.pallas import tpu_sc as plsc

NC = 2
NS = 16
CH = 100
G = 5
EB = 32000


def _fmt_mlp_body(eat_ref, xt_ref, w1t_ref, b1c_ref, w2t_ref, b2c_ref,
                  eye_ref, ef_ref, x0_ref, x1_ref):
    ht = jnp.dot(w1t_ref[...], eat_ref[...], preferred_element_type=jnp.float32)
    ht = jnp.maximum(ht + b1c_ref[...], 0.0)
    eft = jnp.dot(w2t_ref[...], ht, preferred_element_type=jnp.float32)
    eft = eft + b2c_ref[...]
    eye = eye_ref[...]
    br = EB // 8

    def pack_store(tv16, out_ref):
        acc = None
        for j in range(8):
            p = jax.lax.dot_general(
                tv16[:, j * br:(j + 1) * br], eye[16 * j:16 * (j + 1), :],
                (((0,), (0,)), ((), ())), preferred_element_type=jnp.float32)
            acc = p if acc is None else acc + p
        out_ref[...] = acc

    pack_store(eft, ef_ref)
    pack_store(xt_ref[0], x0_ref)
    pack_store(xt_ref[1], x1_ref)


def _fmt_mlp(eat, xt, w1t, b1c, w2t, b2c, n_edges):
    grid = n_edges // EB
    shp = jax.ShapeDtypeStruct((n_edges // 8, 128), jnp.float32)
    return pl.pallas_call(
        _fmt_mlp_body,
        grid=(grid,),
        in_specs=[
            pl.BlockSpec((4, EB), lambda i: (0, i)),
            pl.BlockSpec((2, 16, EB), lambda i: (0, 0, i)),
            pl.BlockSpec((16, 4), lambda i: (0, 0)),
            pl.BlockSpec((16, 1), lambda i: (0, 0)),
            pl.BlockSpec((16, 16), lambda i: (0, 0)),
            pl.BlockSpec((16, 1), lambda i: (0, 0)),
            pl.BlockSpec((128, 128), lambda i: (0, 0)),
        ],
        out_specs=[
            pl.BlockSpec((EB // 8, 128), lambda i: (i, 0)),
            pl.BlockSpec((EB // 8, 128), lambda i: (i, 0)),
            pl.BlockSpec((EB // 8, 128), lambda i: (i, 0)),
        ],
        out_shape=[shp, shp, shp],
    )(eat, xt, w1t, b1c, w2t, b2c, jnp.eye(128, dtype=jnp.float32))


def _make_scatter(n_edges, n_pad):
    n_chunks = n_edges // CH
    ngrp = n_chunks // G // (NC * NS)
    pairs = ngrp // 2
    rpt = n_pad // NS
    mesh = plsc.VectorSubcoreMesh(core_axis_name="c", subcore_axis_name="s")

    @functools.partial(
        pl.kernel,
        compiler_params=pltpu.CompilerParams(use_tc_tiling_on_sc=False),
        out_type=jax.ShapeDtypeStruct((NC, 3, n_pad, 16), jnp.float32),
        mesh=mesh,
        scratch_types=[
            pltpu.VMEM((G, CH), jnp.int32),
            pltpu.VMEM((G, CH), jnp.int32),
            pltpu.VMEM((G, CH, 16), jnp.float32),
            pltpu.VMEM((G, CH, 16), jnp.float32),
            pltpu.VMEM((G, CH, 16), jnp.float32),
            pltpu.VMEM((G, CH, 16), jnp.float32),
            pltpu.VMEM((G, CH, 16), jnp.float32),
            pltpu.VMEM((G, CH, 16), jnp.float32),
            pltpu.VMEM_SHARED((n_pad, 16), jnp.float32),
            pltpu.VMEM_SHARED((n_pad, 16), jnp.float32),
            pltpu.VMEM_SHARED((n_pad, 16), jnp.float32),
            pltpu.SemaphoreType.DMA,
            pltpu.SemaphoreType.DMA,
            pltpu.SemaphoreType.DMA,
            pltpu.SemaphoreType.DMA,
        ],
    )
    def scatter(seg5, ef5, x05, x15, zrows, out,
                ixA, ixB, vefA, vx0A, vx1A, vefB, vx0B, vx1B,
                aef, ax0, ax1, rA, rB, sA, sB):
        c = lax.axis_index("c")
        s = lax.axis_index("s")
        w = c * NS + s
        r0 = s * rpt
        pltpu.sync_copy(zrows, aef.at[pl.ds(r0, rpt)])
        pltpu.sync_copy(zrows, ax0.at[pl.ds(r0, rpt)])
        pltpu.sync_copy(zrows, ax1.at[pl.ds(r0, rpt)])
        plsc.subcore_barrier()

        bufsA = ((vefA, ef5, aef), (vx0A, x05, ax0), (vx1A, x15, ax1))
        bufsB = ((vefB, ef5, aef), (vx0B, x05, ax0), (vx1B, x15, ax1))

        def body(i, carry):
            gA = w * ngrp + 2 * i
            gB = gA + 1
            dA = [pltpu.async_copy(seg5.at[gA], ixA, rA)]
            dA += [pltpu.async_copy(src.at[gA], buf, rA)
                   for buf, src, _ in bufsA]
            dB = [pltpu.async_copy(seg5.at[gB], ixB, rB)]
            dB += [pltpu.async_copy(src.at[gB], buf, rB)
                   for buf, src, _ in bufsB]
            for d in dA:
                d.wait()
            sAd = [pltpu.async_copy(buf.at[b], acc.at[ixA.at[b]], sA, add=True)
                   for b in range(G) for buf, _, acc in bufsA]
            for d in dB:
                d.wait()
            sBd = [pltpu.async_copy(buf.at[b], acc.at[ixB.at[b]], sB, add=True)
                   for b in range(G) for buf, _, acc in bufsB]
            for d in sAd:
                d.wait()
            for d in sBd:
                d.wait()
            return carry

        lax.fori_loop(0, pairs, body, 0)
        plsc.subcore_barrier()
        pltpu.sync_copy(aef.at[pl.ds(r0, rpt)], out.at[c, 0, pl.ds(r0, rpt)])
        pltpu.sync_copy(ax0.at[pl.ds(r0, rpt)], out.at[c, 1, pl.ds(r0, rpt)])
        pltpu.sync_copy(ax1.at[pl.ds(r0, rpt)], out.at[c, 2, pl.ds(r0, rpt)])

    return scatter


def _ff_body(acc_ref, w3a_ref, w3b_ref, b3_ref, w4_ref, b4_ref, out_ref):
    ef = acc_ref[0, 0] + acc_ref[1, 0]
    x0 = acc_ref[0, 1] + acc_ref[1, 1]
    x1 = acc_ref[0, 2] + acc_ref[1, 2]
    w3a = w3a_ref[...]
    w3b = w3b_ref[...]
    ef = ef[:10000]
    x0 = x0[:10000]
    x1 = x1[:10000]
    efw = jnp.dot(ef, w3a, preferred_element_type=jnp.float32) + b3_ref[...]
    for b, xb in ((0, x0), (1, x1)):
        h2 = jnp.maximum(
            efw + jnp.dot(xb, w3b, preferred_element_type=jnp.float32), 0.0)
        out_ref[b] = (jnp.dot(h2, w4_ref[...], preferred_element_type=jnp.float32)
                      + b4_ref[...])


def _final_ff(acc, w3a, w3b, b3, w4, b4, n_pad):
    return pl.pallas_call(
        _ff_body,
        in_specs=[pl.BlockSpec(memory_space=pltpu.VMEM)] * 6,
        out_specs=pl.BlockSpec(memory_space=pltpu.VMEM),
        out_shape=jax.ShapeDtypeStruct((2, 10000, 32), jnp.float32),
    )(acc, w3a, w3b, b3, w4, b4)


def kernel(x, edge_index, edge_attr, w1, b1, w2, b2, w3, b3, w4, b4):
    B, E, DX = x.shape
    n_rec = 10000
    n_pad = 10240

    eat = edge_attr.T
    xt = jnp.transpose(x, (0, 2, 1))
    ef, x0r, x1r = _fmt_mlp(eat, xt, w1.T, b1[:, None], w2.T, b2[:, None], E)

    nq = E // CH // G
    seg_pk = edge_index[1].reshape(E // EB, 8, EB // 8)
    seg_pk = seg_pk.transpose(0, 2, 1).reshape(-1)
    seg5 = seg_pk.reshape(nq, G, CH)
    scatter = _make_scatter(E, n_pad)
    zrows = jnp.zeros((n_pad // NS, 16), jnp.float32)
    acc = scatter(seg5,
                  ef.reshape(nq, G, CH, 16),
                  x0r.reshape(nq, G, CH, 16),
                  x1r.reshape(nq, G, CH, 16),
                  zrows)

    w3a = w3[:16]
    w3b = w3[16:]
    out = _final_ff(acc, w3a, w3b, b3[None, :], w4, b4[None, :], n_pad)
    return out
